# Initial kernel scaffold; baseline (speedup 1.0000x reference)
import jax
import jax.numpy as jnp
from jax.experimental import pallas as pl


def kernel(z, codes):
    raise NotImplementedError("write your pallas kernel here")


# speedup vs baseline: 3.4274x; 3.4274x over previous
import functools

import jax
import jax.numpy as jnp
from jax import lax
from jax.experimental import pallas as pl


def _encoder_body(x_ref, c_ref, soft_ref, hard_ref, idx_ref):
    x = x_ref[0]
    c = c_ref[0]
    N = x.shape[0]
    K = c.shape[0]
    CD = x.shape[1]

    dot = lax.dot_general(x, c, (((1,), (1,)), ((), ())),
                          precision=lax.Precision.HIGHEST,
                          preferred_element_type=jnp.float32)
    cn = lax.dot_general(jnp.ones((1, CD), jnp.float32), c * c,
                         (((1,), (1,)), ((), ())),
                         precision=lax.Precision.HIGHEST,
                         preferred_element_type=jnp.float32)
    xn = jnp.sum(x * x, axis=1, keepdims=True)
    d2 = xn - 2.0 * dot + cn
    d = jnp.sqrt(jnp.maximum(d2, 0.0))

    dmin = jnp.min(d, axis=1, keepdims=True)
    kio = lax.broadcasted_iota(jnp.int32, (N, K), 1)
    idx = jnp.min(jnp.where(d == dmin, kio, K), axis=1)
    idx_ref[0, 0] = idx

    p = jnp.exp(dmin - d)
    s = jnp.sum(p, axis=1, keepdims=True)
    soft = lax.dot_general(p, c, (((1,), (0,)), ((), ())),
                           precision=lax.Precision.HIGHEST,
                           preferred_element_type=jnp.float32)
    soft_ref[0] = soft / s

    onehot = jnp.where(kio == idx[:, None], 1.0, 0.0).astype(jnp.float32)
    hard = lax.dot_general(onehot, c, (((1,), (0,)), ((), ())),
                           precision=lax.Precision.HIGHEST,
                           preferred_element_type=jnp.float32)
    hard_ref[0] = hard


@jax.jit
def _encode(zt, codes):
    L, N, CD = zt.shape
    _, K, _ = codes.shape
    soft, hard, idx = pl.pallas_call(
        _encoder_body,
        grid=(L,),
        in_specs=[
            pl.BlockSpec((1, N, CD), lambda l: (l, 0, 0)),
            pl.BlockSpec((1, K, CD), lambda l: (l, 0, 0)),
        ],
        out_specs=[
            pl.BlockSpec((1, N, CD), lambda l: (l, 0, 0)),
            pl.BlockSpec((1, N, CD), lambda l: (l, 0, 0)),
            pl.BlockSpec((1, 1, N), lambda l: (l, 0, 0)),
        ],
        out_shape=[
            jax.ShapeDtypeStruct((L, N, CD), jnp.float32),
            jax.ShapeDtypeStruct((L, N, CD), jnp.float32),
            jax.ShapeDtypeStruct((L, 1, N), jnp.int32),
        ],
    )(zt, codes)
    return soft, hard, idx


def kernel(z, codes):
    B, C, H, Wd = z.shape
    L, K, CD = codes.shape
    N = B * H * Wd
    h = jnp.transpose(z, (0, 2, 3, 1)).reshape(N, L, CD)
    zt = jnp.transpose(h, (1, 0, 2))

    soft, hard, idx = _encode(zt, codes)

    soft_symbols = jnp.transpose(soft, (1, 0, 2)).reshape(B, H, Wd, C)
    hard_symbols = jnp.transpose(hard, (1, 0, 2)).reshape(B, H, Wd, C)
    idxes = jnp.transpose(idx.reshape(L, N), (1, 0)).reshape(B, H, Wd, L)
    return (soft_symbols, hard_symbols, idxes)
